# baseline (speedup 1.0000x reference)
import jax
import jax.numpy as jnp
from jax import lax
from jax.experimental import pallas as pl
from jax.experimental.pallas import tpu as pltpu
from jax.experimental.pallas import tpu_sc as plsc

BATCH = 16384
EMBED_DIM = 64
ROW_PAD = 128
NUM_CORES = 2
NUM_SUBCORES = 16
NUM_WORKERS = NUM_CORES * NUM_SUBCORES
B_PER_W = BATCH // NUM_WORKERS
HALF = B_PER_W // 4
LANES = 16
VECS_PER_ROW = EMBED_DIM // LANES


def _body(node_hbm, rel_hbm, head_hbm, relids_hbm, tail_hbm, out_hbm,
          vidx_h, vidx_r, vidx_t,
          h_buf, r_buf, t_buf, o_buf,
          sem_h, sem_r, sem_t):
    wid = lax.axis_index("s") * NUM_CORES + lax.axis_index("c")
    base = wid * B_PER_W

    pltpu.sync_copy(head_hbm.at[pl.ds(base, B_PER_W)],
                    vidx_h.at[pl.ds(0, B_PER_W)])
    pltpu.sync_copy(relids_hbm.at[pl.ds(base, B_PER_W)],
                    vidx_r.at[pl.ds(0, B_PER_W)])
    pltpu.sync_copy(tail_hbm.at[pl.ds(base, B_PER_W)],
                    vidx_t.at[pl.ds(0, B_PER_W)])

    for half in range(4):
        off = half * HALF

        def issue_body(i, carry):
            hid = vidx_h[pl.ds(off + i, LANES)][0]
            rid = vidx_r[pl.ds(off + i, LANES)][0]
            tid = vidx_t[pl.ds(off + i, LANES)][0]
            pltpu.async_copy(node_hbm.at[pl.ds(hid, 1)],
                             h_buf.at[pl.ds(i, 1)], sem_h)
            pltpu.async_copy(rel_hbm.at[pl.ds(rid, 1)],
                             r_buf.at[pl.ds(i, 1)], sem_r)
            pltpu.async_copy(node_hbm.at[pl.ds(tid, 1)],
                             t_buf.at[pl.ds(i, 1)], sem_t)
            return carry

        lax.fori_loop(0, HALF, issue_body, 0)

        pltpu.make_async_copy(node_hbm.at[pl.ds(0, HALF)], h_buf,
                              sem_h).wait()
        pltpu.make_async_copy(rel_hbm.at[pl.ds(0, HALF)], r_buf,
                              sem_r).wait()
        pltpu.make_async_copy(node_hbm.at[pl.ds(0, HALF)], t_buf,
                              sem_t).wait()

        def row_body(i, carry):
            for j in range(VECS_PER_ROW):
                sl = pl.ds(j * LANES, LANES)
                o_buf[i, sl] = h_buf[i, sl] + r_buf[i, sl] + t_buf[i, sl]
            return carry

        lax.fori_loop(0, HALF, row_body, 0)

        pltpu.sync_copy(o_buf, out_hbm.at[pl.ds(base + off, HALF)])


@jax.jit
def kernel(head_ids, rel_ids, tail_ids, node_embeddings, rel_weight):
    mesh = plsc.VectorSubcoreMesh(core_axis_name="c", subcore_axis_name="s",
                                  num_cores=NUM_CORES,
                                  num_subcores=NUM_SUBCORES)
    k = pl.kernel(
        _body,
        out_type=jax.ShapeDtypeStruct((BATCH, ROW_PAD), jnp.float32),
        mesh=mesh,
        compiler_params=pltpu.CompilerParams(needs_layout_passes=False),
        scratch_types=[
            pltpu.VMEM((B_PER_W + LANES,), jnp.int32),
            pltpu.VMEM((B_PER_W + LANES,), jnp.int32),
            pltpu.VMEM((B_PER_W + LANES,), jnp.int32),
            pltpu.VMEM((HALF, EMBED_DIM), jnp.float32),
            pltpu.VMEM((HALF, EMBED_DIM), jnp.float32),
            pltpu.VMEM((HALF, EMBED_DIM), jnp.float32),
            pltpu.VMEM((HALF, ROW_PAD), jnp.float32),
            pltpu.SemaphoreType.DMA,
            pltpu.SemaphoreType.DMA,
            pltpu.SemaphoreType.DMA,
        ],
    )
    out_pad = k(node_embeddings, rel_weight, head_ids, rel_ids, tail_ids)
    return out_pad[:, :EMBED_DIM]
